# Initial kernel scaffold; baseline (speedup 1.0000x reference)
import functools

import jax
import jax.numpy as jnp
from jax import lax
from jax.experimental import pallas as pl
from jax.experimental.pallas import tpu as pltpu
from jax.experimental.pallas import tpu_sc as plsc

N = 10000
D = 128
E = 320000

C = 128
NCHUNKS = E // C
NW = 32
L = 16

BM = 512
BN = 512


def _w_body(beta_ref, a_ref, b_ref, w_ref):
    a = a_ref[...]
    b = b_ref[...]
    g = jnp.dot(a, b, preferred_element_type=jnp.float32)
    na = jnp.sqrt(jnp.sum(a * a, axis=1, keepdims=True))
    nb = jnp.sqrt(jnp.sum(b * b, axis=0, keepdims=True))
    cos = g / (na * nb + 1e-7)
    w_ref[...] = jnp.exp(beta_ref[0] * cos)


def _build_w(x, xt, beta):
    grid = (N // BM, N // BN)
    return pl.pallas_call(
        _w_body,
        grid=grid,
        in_specs=[
            pl.BlockSpec(memory_space=pltpu.SMEM),
            pl.BlockSpec((BM, D), lambda i, j: (i, 0)),
            pl.BlockSpec((D, BN), lambda i, j: (0, j)),
        ],
        out_specs=pl.BlockSpec((BM, BN), lambda i, j: (i, j)),
        out_shape=jax.ShapeDtypeStruct((N, N), jnp.float32),
    )(jnp.reshape(beta, (1,)), x, xt)


def _sc_a_body(wflat, rowh, colh, zerosn,
               wbuf, denom_p,
               row_v, col_v, lin_v, w_v,
               denom_sh, sem):
    cid = lax.axis_index("c")
    sid = lax.axis_index("s")
    wid = cid * 16 + sid

    @pl.when(sid == 0)
    def _init():
        pltpu.sync_copy(zerosn, denom_sh)

    plsc.subcore_barrier()

    nch = (NCHUNKS - wid + (NW - 1)) // NW

    def chunk(g, carry):
        base = (wid + g * NW) * C
        pltpu.sync_copy(rowh.at[pl.ds(base, C)], row_v)
        pltpu.sync_copy(colh.at[pl.ds(base, C)], col_v)
        for j in range(C // L):
            s = pl.ds(j * L, L)
            lin_v[s] = row_v[s] * N + col_v[s]
        pltpu.async_copy(wflat.at[lin_v], w_v, sem).wait()
        pltpu.sync_copy(w_v, wbuf.at[pl.ds(base, C)])
        pltpu.sync_copy(w_v, denom_sh.at[row_v], add=True)
        return carry

    lax.fori_loop(0, nch, chunk, 0)
    plsc.subcore_barrier()

    @pl.when(sid == 0)
    def _out():
        pltpu.sync_copy(denom_sh, denom_p.at[cid])


def _sc_phase_a(wflat, row, col):
    mesh = plsc.VectorSubcoreMesh(core_axis_name="c", subcore_axis_name="s")
    zerosn = jnp.zeros((N,), jnp.float32)
    return pl.kernel(
        _sc_a_body,
        out_type=(
            jax.ShapeDtypeStruct((E,), jnp.float32),
            jax.ShapeDtypeStruct((2, N), jnp.float32),
        ),
        mesh=mesh,
        scratch_types=[
            pltpu.VMEM((C,), jnp.int32),
            pltpu.VMEM((C,), jnp.int32),
            pltpu.VMEM((C,), jnp.int32),
            pltpu.VMEM((C,), jnp.float32),
            pltpu.VMEM_SHARED((N,), jnp.float32),
            pltpu.SemaphoreType.DMA,
        ],
    )(wflat, row, col, zerosn)


def _sc_b_body(xh, rowh, colh, wbufh, dpf, zerosnd,
               out_p,
               row_v, col_v, rp_v, w_v, d0_v, d1_v, p_v, a_v,
               out_sh, sem):
    cid = lax.axis_index("c")
    sid = lax.axis_index("s")
    wid = cid * 16 + sid

    @pl.when(sid == 0)
    def _init():
        pltpu.sync_copy(zerosnd, out_sh)

    plsc.subcore_barrier()

    nch = (NCHUNKS - wid + (NW - 1)) // NW

    def chunk(g, carry):
        base = (wid + g * NW) * C
        pltpu.sync_copy(rowh.at[pl.ds(base, C)], row_v)
        pltpu.sync_copy(colh.at[pl.ds(base, C)], col_v)
        pltpu.sync_copy(wbufh.at[pl.ds(base, C)], w_v)
        for j in range(C // L):
            s = pl.ds(j * L, L)
            rp_v[s] = row_v[s] + N
        pltpu.async_copy(dpf.at[row_v], d0_v, sem).wait()
        pltpu.async_copy(dpf.at[rp_v], d1_v, sem).wait()
        for j in range(C // L):
            s = pl.ds(j * L, L)
            p_v[s] = w_v[s] / (d0_v[s] + d1_v[s] + 1e-16)
        pltpu.async_copy(xh.at[row_v], a_v, sem).wait()

        def rowscale(j, c2):
            pv = plsc.load_gather(p_v, [jnp.full((L,), j, jnp.int32)])
            for k in range(D // L):
                s = pl.ds(k * L, L)
                a_v[j, s] = a_v[j, s] * pv
            return c2

        lax.fori_loop(0, C, rowscale, 0)
        pltpu.sync_copy(a_v, out_sh.at[col_v], add=True)
        return carry

    lax.fori_loop(0, nch, chunk, 0)
    plsc.subcore_barrier()

    @pl.when(sid == 0)
    def _out():
        pltpu.sync_copy(out_sh, out_p.at[cid])


def _sc_phase_b(x, row, col, wbuf, dpf):
    mesh = plsc.VectorSubcoreMesh(core_axis_name="c", subcore_axis_name="s")
    zerosnd = jnp.zeros((N, D), jnp.float32)
    return pl.kernel(
        _sc_b_body,
        out_type=jax.ShapeDtypeStruct((2, N, D), jnp.float32),
        mesh=mesh,
        scratch_types=[
            pltpu.VMEM((C,), jnp.int32),
            pltpu.VMEM((C,), jnp.int32),
            pltpu.VMEM((C,), jnp.int32),
            pltpu.VMEM((C,), jnp.float32),
            pltpu.VMEM((C,), jnp.float32),
            pltpu.VMEM((C,), jnp.float32),
            pltpu.VMEM((C,), jnp.float32),
            pltpu.VMEM((C, D), jnp.float32),
            pltpu.VMEM_SHARED((N, D), jnp.float32),
            pltpu.SemaphoreType.DMA,
        ],
    )(x, row, col, wbuf, dpf, zerosnd)


def _sum_body(p_ref, o_ref):
    o_ref[...] = p_ref[0] + p_ref[1]


def _sum_partials(out_p):
    rows = 1250
    return pl.pallas_call(
        _sum_body,
        grid=(N // rows,),
        in_specs=[pl.BlockSpec((2, rows, D), lambda i: (0, i, 0))],
        out_specs=pl.BlockSpec((rows, D), lambda i: (i, 0)),
        out_shape=jax.ShapeDtypeStruct((N, D), jnp.float32),
    )(out_p)


def kernel(x, edge_index, beta):
    row = edge_index[0]
    col = edge_index[1]
    w = _build_w(x, x.T, beta)
    wbuf, denom_p = _sc_phase_a(w.reshape(N * N), row, col)
    out_p = _sc_phase_b(x, row, col, wbuf, denom_p.reshape(2 * N))
    return _sum_partials(out_p)


# speedup vs baseline: 5.5884x; 5.5884x over previous
import functools

import jax
import jax.numpy as jnp
from jax import lax
from jax.experimental import pallas as pl
from jax.experimental.pallas import tpu as pltpu
from jax.experimental.pallas import tpu_sc as plsc

N = 10000
D = 128
E = 320000

C = 128
NCHUNKS = E // C
NW = 32
L = 16

NP = 10240
BM = 512
BN = 512


def _w_body(beta_ref, a_ref, b_ref, w_ref):
    a = a_ref[...]
    b = b_ref[...]
    g = jnp.dot(a, b, preferred_element_type=jnp.float32)
    na = jnp.sqrt(jnp.sum(a * a, axis=1, keepdims=True))
    nb = jnp.sqrt(jnp.sum(b * b, axis=0, keepdims=True))
    cos = g / (na * nb + 1e-7)
    w_ref[...] = jnp.exp(beta_ref[0] * cos)


def _build_w(x, xt, beta):
    grid = (NP // BM, NP // BN)
    return pl.pallas_call(
        _w_body,
        grid=grid,
        in_specs=[
            pl.BlockSpec(memory_space=pltpu.SMEM),
            pl.BlockSpec((BM, D), lambda i, j: (i, 0)),
            pl.BlockSpec((D, BN), lambda i, j: (0, j)),
        ],
        out_specs=pl.BlockSpec((BM, BN), lambda i, j: (i, j)),
        out_shape=jax.ShapeDtypeStruct((NP, NP), jnp.float32),
    )(jnp.reshape(beta, (1,)), x, xt)


def _sc_a_body(wflat, rowh, colh, zerosn,
               wbuf, denom_p,
               row_v, col_v, lin_v, w_v,
               denom_sh, sem):
    cid = lax.axis_index("c")
    sid = lax.axis_index("s")
    wid = cid * 16 + sid

    @pl.when(sid == 0)
    def _init():
        pltpu.sync_copy(zerosn, denom_sh)

    plsc.subcore_barrier()

    nch = (NCHUNKS - wid + (NW - 1)) // NW

    def chunk(g, carry):
        base = (wid + g * NW) * C
        pltpu.sync_copy(rowh.at[pl.ds(base, C)], row_v)
        pltpu.sync_copy(colh.at[pl.ds(base, C)], col_v)
        for j in range(C // L):
            s = pl.ds(j * L, L)
            lin_v[s] = row_v[s] * NP + col_v[s]
        pltpu.async_copy(wflat.at[lin_v], w_v, sem).wait()
        pltpu.sync_copy(w_v, wbuf.at[pl.ds(base, C)])
        pltpu.sync_copy(w_v, denom_sh.at[row_v], add=True)
        return carry

    lax.fori_loop(0, nch, chunk, 0)
    plsc.subcore_barrier()

    @pl.when(sid == 0)
    def _out():
        pltpu.sync_copy(denom_sh, denom_p.at[cid])


def _sc_phase_a(wflat, row, col):
    mesh = plsc.VectorSubcoreMesh(core_axis_name="c", subcore_axis_name="s")
    zerosn = jnp.zeros((N,), jnp.float32)
    return pl.kernel(
        _sc_a_body,
        out_type=(
            jax.ShapeDtypeStruct((E,), jnp.float32),
            jax.ShapeDtypeStruct((2, N), jnp.float32),
        ),
        mesh=mesh,
        scratch_types=[
            pltpu.VMEM((C,), jnp.int32),
            pltpu.VMEM((C,), jnp.int32),
            pltpu.VMEM((C,), jnp.int32),
            pltpu.VMEM((C,), jnp.float32),
            pltpu.VMEM_SHARED((N,), jnp.float32),
            pltpu.SemaphoreType.DMA,
        ],
    )(wflat, row, col, zerosn)


def _sc_b_body(xh, rowh, colh, wbufh, dpf, zerosnd,
               out_p,
               row_v, col_v, rp_v, w_v, d0_v, d1_v, p_v, a_v,
               out_sh, sem):
    cid = lax.axis_index("c")
    sid = lax.axis_index("s")
    wid = cid * 16 + sid

    @pl.when(sid == 0)
    def _init():
        pltpu.sync_copy(zerosnd, out_sh)

    plsc.subcore_barrier()

    nch = (NCHUNKS - wid + (NW - 1)) // NW

    def chunk(g, carry):
        base = (wid + g * NW) * C
        pltpu.sync_copy(rowh.at[pl.ds(base, C)], row_v)
        pltpu.sync_copy(colh.at[pl.ds(base, C)], col_v)
        pltpu.sync_copy(wbufh.at[pl.ds(base, C)], w_v)
        for j in range(C // L):
            s = pl.ds(j * L, L)
            rp_v[s] = row_v[s] + N
        pltpu.async_copy(dpf.at[row_v], d0_v, sem).wait()
        pltpu.async_copy(dpf.at[rp_v], d1_v, sem).wait()
        for j in range(C // L):
            s = pl.ds(j * L, L)
            p_v[s] = w_v[s] / (d0_v[s] + d1_v[s] + 1e-16)
        pltpu.async_copy(xh.at[row_v], a_v, sem).wait()

        def rowscale16(m, c2):
            p16 = p_v[pl.ds(m * L, L)]
            for jj in range(L):
                pv = p16[jj]
                j = m * L + jj
                for k in range(D // L):
                    s = pl.ds(k * L, L)
                    a_v[j, s] = a_v[j, s] * pv
            return c2

        lax.fori_loop(0, C // L, rowscale16, 0)
        pltpu.sync_copy(a_v, out_sh.at[col_v], add=True)
        return carry

    lax.fori_loop(0, nch, chunk, 0)
    plsc.subcore_barrier()

    @pl.when(sid == 0)
    def _out():
        pltpu.sync_copy(out_sh, out_p.at[cid])


def _sc_phase_b(x, row, col, wbuf, dpf):
    mesh = plsc.VectorSubcoreMesh(core_axis_name="c", subcore_axis_name="s")
    zerosnd = jnp.zeros((N, D), jnp.float32)
    return pl.kernel(
        _sc_b_body,
        out_type=jax.ShapeDtypeStruct((2, N, D), jnp.float32),
        mesh=mesh,
        scratch_types=[
            pltpu.VMEM((C,), jnp.int32),
            pltpu.VMEM((C,), jnp.int32),
            pltpu.VMEM((C,), jnp.int32),
            pltpu.VMEM((C,), jnp.float32),
            pltpu.VMEM((C,), jnp.float32),
            pltpu.VMEM((C,), jnp.float32),
            pltpu.VMEM((C,), jnp.float32),
            pltpu.VMEM((C, D), jnp.float32),
            pltpu.VMEM_SHARED((N, D), jnp.float32),
            pltpu.SemaphoreType.DMA,
        ],
    )(x, row, col, wbuf, dpf, zerosnd)


def _sum_body(p_ref, o_ref):
    o_ref[...] = p_ref[0] + p_ref[1]


def _sum_partials(out_p):
    rows = 2000
    return pl.pallas_call(
        _sum_body,
        grid=(N // rows,),
        in_specs=[pl.BlockSpec((2, rows, D), lambda i: (0, i, 0))],
        out_specs=pl.BlockSpec((rows, D), lambda i: (i, 0)),
        out_shape=jax.ShapeDtypeStruct((N, D), jnp.float32),
    )(out_p)


def kernel(x, edge_index, beta):
    row = edge_index[0]
    col = edge_index[1]
    xp = jnp.pad(x, ((0, NP - N), (0, 0)))
    w = _build_w(xp, xp.T, beta)
    wbuf, denom_p = _sc_phase_a(w.reshape(NP * NP), row, col)
    out_p = _sc_phase_b(x, row, col, wbuf, denom_p.reshape(2 * N))
    return _sum_partials(out_p)


# speedup vs baseline: 6.1126x; 1.0938x over previous
import functools

import jax
import jax.numpy as jnp
from jax import lax
from jax.experimental import pallas as pl
from jax.experimental.pallas import tpu as pltpu
from jax.experimental.pallas import tpu_sc as plsc

N = 10000
D = 128
E = 320000

C = 128
NCHUNKS = E // C
NW = 32
L = 16

NP = 10240
BM = 1024
BN = 128


def _norm_body(x_ref, o_ref):
    xv = x_ref[...]
    s2 = jnp.sum(xv * xv, axis=1, keepdims=True)
    o_ref[...] = xv * lax.rsqrt(s2 + 1e-30)


def _normalize(x):
    return pl.pallas_call(
        _norm_body,
        out_shape=jax.ShapeDtypeStruct((N, D), jnp.float32),
    )(x)


def _w_body(beta_ref, a_ref, b_ref, w_ref):
    a = a_ref[...]
    b = b_ref[...]
    dn = (((1,), (1,)), ((), ()))
    cos = lax.dot_general(a, b, dn, preferred_element_type=jnp.float32)
    w_ref[...] = jnp.exp(beta_ref[0] * cos)[None]


def _build_w(x, beta):
    grid = (NP // BM, NP // BN)
    return pl.pallas_call(
        _w_body,
        grid=grid,
        in_specs=[
            pl.BlockSpec(memory_space=pltpu.SMEM),
            pl.BlockSpec((BM, D), lambda i, j: (i, 0)),
            pl.BlockSpec((BN, D), lambda i, j: (j, 0)),
        ],
        out_specs=pl.BlockSpec((1, BM, BN), lambda i, j: (j, i, 0)),
        out_shape=jax.ShapeDtypeStruct((NP // BN, NP, BN), jnp.float32),
    )(jnp.reshape(beta, (1,)), x, x)


def _sc_a_body(wflat, rowh, colh, zerosn,
               wbuf, denom_p,
               row_v, col_v, lin_v, w_v,
               denom_sh, sem):
    cid = lax.axis_index("c")
    sid = lax.axis_index("s")
    wid = cid * 16 + sid

    @pl.when(sid == 0)
    def _init():
        pltpu.sync_copy(zerosn, denom_sh)

    plsc.subcore_barrier()

    nch = (NCHUNKS - wid + (NW - 1)) // NW

    def chunk(g, carry):
        base = (wid + g * NW) * C
        pltpu.sync_copy(rowh.at[pl.ds(base, C)], row_v)
        pltpu.sync_copy(colh.at[pl.ds(base, C)], col_v)
        for j in range(C // L):
            s = pl.ds(j * L, L)
            c = col_v[s]
            lin_v[s] = ((c >> 7) * (NP * 128) + (row_v[s] << 7)
                        + (c & 127))
        pltpu.async_copy(wflat.at[lin_v], w_v, sem).wait()
        pltpu.sync_copy(w_v, wbuf.at[pl.ds(base, C)])
        pltpu.sync_copy(w_v, denom_sh.at[row_v], add=True)
        return carry

    lax.fori_loop(0, nch, chunk, 0)
    plsc.subcore_barrier()

    @pl.when(sid == 0)
    def _out():
        pltpu.sync_copy(denom_sh, denom_p.at[cid])


def _sc_phase_a(wflat, row, col):
    mesh = plsc.VectorSubcoreMesh(core_axis_name="c", subcore_axis_name="s")
    zerosn = jnp.zeros((N,), jnp.float32)
    return pl.kernel(
        _sc_a_body,
        out_type=(
            jax.ShapeDtypeStruct((E,), jnp.float32),
            jax.ShapeDtypeStruct((2, N), jnp.float32),
        ),
        mesh=mesh,
        scratch_types=[
            pltpu.VMEM((C,), jnp.int32),
            pltpu.VMEM((C,), jnp.int32),
            pltpu.VMEM((C,), jnp.int32),
            pltpu.VMEM((C,), jnp.float32),
            pltpu.VMEM_SHARED((N,), jnp.float32),
            pltpu.SemaphoreType.DMA,
        ],
    )(wflat, row, col, zerosn)


def _sc_b_body(xh, rowh, colh, wbufh, dpf, zerosnd,
               out_p,
               row_v, col_v, rp_v, w_v, d0_v, d1_v, p_v, a_v,
               out_sh, sem):
    cid = lax.axis_index("c")
    sid = lax.axis_index("s")
    wid = cid * 16 + sid

    @pl.when(sid == 0)
    def _init():
        pltpu.sync_copy(zerosnd, out_sh)

    plsc.subcore_barrier()

    nch = (NCHUNKS - wid + (NW - 1)) // NW

    def chunk(g, carry):
        base = (wid + g * NW) * C
        pltpu.sync_copy(rowh.at[pl.ds(base, C)], row_v)
        pltpu.sync_copy(colh.at[pl.ds(base, C)], col_v)
        pltpu.sync_copy(wbufh.at[pl.ds(base, C)], w_v)
        for j in range(C // L):
            s = pl.ds(j * L, L)
            rp_v[s] = row_v[s] + N
        pltpu.async_copy(dpf.at[row_v], d0_v, sem).wait()
        pltpu.async_copy(dpf.at[rp_v], d1_v, sem).wait()
        for j in range(C // L):
            s = pl.ds(j * L, L)
            p_v[s] = w_v[s] / (d0_v[s] + d1_v[s] + 1e-16)
        pltpu.async_copy(xh.at[row_v], a_v, sem).wait()

        def rowscale16(m, c2):
            p16 = p_v[pl.ds(m * L, L)]
            for jj in range(L):
                pv = p16[jj]
                j = m * L + jj
                for k in range(D // L):
                    s = pl.ds(k * L, L)
                    a_v[j, s] = a_v[j, s] * pv
            return c2

        lax.fori_loop(0, C // L, rowscale16, 0)
        pltpu.sync_copy(a_v, out_sh.at[col_v], add=True)
        return carry

    lax.fori_loop(0, nch, chunk, 0)
    plsc.subcore_barrier()

    @pl.when(sid == 0)
    def _out():
        pltpu.sync_copy(out_sh, out_p.at[cid])


def _sc_phase_b(x, row, col, wbuf, dpf):
    mesh = plsc.VectorSubcoreMesh(core_axis_name="c", subcore_axis_name="s")
    zerosnd = jnp.zeros((N, D), jnp.float32)
    return pl.kernel(
        _sc_b_body,
        out_type=jax.ShapeDtypeStruct((2, N, D), jnp.float32),
        mesh=mesh,
        scratch_types=[
            pltpu.VMEM((C,), jnp.int32),
            pltpu.VMEM((C,), jnp.int32),
            pltpu.VMEM((C,), jnp.int32),
            pltpu.VMEM((C,), jnp.float32),
            pltpu.VMEM((C,), jnp.float32),
            pltpu.VMEM((C,), jnp.float32),
            pltpu.VMEM((C,), jnp.float32),
            pltpu.VMEM((C, D), jnp.float32),
            pltpu.VMEM_SHARED((N, D), jnp.float32),
            pltpu.SemaphoreType.DMA,
        ],
    )(x, row, col, wbuf, dpf, zerosnd)


def _sum_body(p_ref, o_ref):
    o_ref[...] = p_ref[0] + p_ref[1]


def _sum_partials(out_p):
    rows = 2000
    return pl.pallas_call(
        _sum_body,
        grid=(N // rows,),
        in_specs=[pl.BlockSpec((2, rows, D), lambda i: (0, i, 0))],
        out_specs=pl.BlockSpec((rows, D), lambda i: (i, 0)),
        out_shape=jax.ShapeDtypeStruct((N, D), jnp.float32),
    )(out_p)


def kernel(x, edge_index, beta):
    row = edge_index[0]
    col = edge_index[1]
    xnp = jnp.pad(_normalize(x), ((0, NP - N), (0, 0)))
    w = _build_w(xnp, beta)
    wbuf, denom_p = _sc_phase_a(w.reshape(NP * NP), row, col)
    out_p = _sc_phase_b(x, row, col, wbuf, denom_p.reshape(2 * N))
    return _sum_partials(out_p)


# speedup vs baseline: 11.7792x; 1.9270x over previous
import functools

import jax
import jax.numpy as jnp
from jax import lax
from jax.experimental import pallas as pl
from jax.experimental.pallas import tpu as pltpu
from jax.experimental.pallas import tpu_sc as plsc

N = 10000
D = 128
E = 320000

C = 128
NCHUNKS = E // C
NW = 32
L = 16

NP = 10240
BM = 2048
BN = 128


def _norm_body(x_ref, o_ref):
    xv = x_ref[...]
    s2 = jnp.sum(xv * xv, axis=1, keepdims=True)
    o_ref[...] = xv * lax.rsqrt(s2 + 1e-30)


def _normalize(x):
    return pl.pallas_call(
        _norm_body,
        out_shape=jax.ShapeDtypeStruct((N, D), jnp.float32),
    )(x)


def _w_body(beta_ref, a_ref, b_ref, w_ref):
    a = a_ref[...]
    b = b_ref[...]
    dn = (((1,), (1,)), ((), ()))
    cos = lax.dot_general(a, b, dn, preferred_element_type=jnp.float32)
    w_ref[...] = jnp.exp(beta_ref[0] * cos)[None]


def _w_ij(t):
    i = ((t >= 80).astype(jnp.int32) + (t >= 144).astype(jnp.int32)
         + (t >= 192).astype(jnp.int32) + (t >= 224).astype(jnp.int32))
    base = (jnp.where(i >= 1, 80, 0) + jnp.where(i >= 2, 64, 0)
            + jnp.where(i >= 3, 48, 0) + jnp.where(i >= 4, 32, 0))
    j = t - base + 16 * i
    return i, j


N_WBLOCKS = 240


def _build_w(x, beta):
    return pl.pallas_call(
        _w_body,
        grid=(N_WBLOCKS,),
        in_specs=[
            pl.BlockSpec(memory_space=pltpu.SMEM),
            pl.BlockSpec((BM, D), lambda t: (_w_ij(t)[0], 0)),
            pl.BlockSpec((BN, D), lambda t: (_w_ij(t)[1], 0)),
        ],
        out_specs=pl.BlockSpec(
            (1, BM, BN), lambda t: (_w_ij(t)[1], _w_ij(t)[0], 0)),
        out_shape=jax.ShapeDtypeStruct((NP // BN, NP, BN), jnp.float32),
    )(jnp.reshape(beta, (1,)), x, x)


def _sc_a_body(wflat, rowh, colh, zerosn,
               wbuf, denom_p,
               row_v, col_v, lin_v, w_v,
               denom_sh, sem):
    cid = lax.axis_index("c")
    sid = lax.axis_index("s")
    wid = cid * 16 + sid

    @pl.when(sid == 0)
    def _init():
        pltpu.sync_copy(zerosn, denom_sh)

    plsc.subcore_barrier()

    nch = (NCHUNKS - wid + (NW - 1)) // NW

    def chunk(g, carry):
        base = (wid + g * NW) * C
        pltpu.sync_copy(rowh.at[pl.ds(base, C)], row_v)
        pltpu.sync_copy(colh.at[pl.ds(base, C)], col_v)
        for j in range(C // L):
            s = pl.ds(j * L, L)
            r = row_v[s]
            c = col_v[s]
            rm = jnp.minimum(r, c)
            cm = jnp.maximum(r, c)
            lin_v[s] = ((cm >> 7) * (NP * 128) + (rm << 7)
                        + (cm & 127))
        pltpu.async_copy(wflat.at[lin_v], w_v, sem).wait()
        pltpu.sync_copy(w_v, wbuf.at[pl.ds(base, C)])
        pltpu.sync_copy(w_v, denom_sh.at[row_v], add=True)
        return carry

    lax.fori_loop(0, nch, chunk, 0)
    plsc.subcore_barrier()

    @pl.when(sid == 0)
    def _out():
        pltpu.sync_copy(denom_sh, denom_p.at[cid])


def _sc_phase_a(wflat, row, col):
    mesh = plsc.VectorSubcoreMesh(core_axis_name="c", subcore_axis_name="s")
    zerosn = jnp.zeros((N,), jnp.float32)
    return pl.kernel(
        _sc_a_body,
        out_type=(
            jax.ShapeDtypeStruct((E,), jnp.float32),
            jax.ShapeDtypeStruct((2, N), jnp.float32),
        ),
        mesh=mesh,
        scratch_types=[
            pltpu.VMEM((C,), jnp.int32),
            pltpu.VMEM((C,), jnp.int32),
            pltpu.VMEM((C,), jnp.int32),
            pltpu.VMEM((C,), jnp.float32),
            pltpu.VMEM_SHARED((N,), jnp.float32),
            pltpu.SemaphoreType.DMA,
        ],
    )(wflat, row, col, zerosn)


def _dinv_body(p_ref, o_ref):
    o_ref[...] = 1.0 / (p_ref[0] + p_ref[1] + 1e-16)


def _dinv(denom_p):
    return pl.pallas_call(
        _dinv_body,
        out_shape=jax.ShapeDtypeStruct((N,), jnp.float32),
    )(denom_p)


def _sc_b_body(xh, rowh, colh, wbufh, dinvh, zerosnd,
               out_p,
               row_v, col_v, w_v, di_v, p_v, a_v,
               out_sh, sem_ld, sem_d, sem_a):
    cid = lax.axis_index("c")
    sid = lax.axis_index("s")
    wid = cid * 16 + sid

    @pl.when(sid == 0)
    def _init():
        pltpu.sync_copy(zerosnd, out_sh)

    plsc.subcore_barrier()

    nch = (NCHUNKS - wid + (NW - 1)) // NW

    def issue_loads(g, b):
        base = (wid + g * NW) * C
        pltpu.async_copy(rowh.at[pl.ds(base, C)], row_v.at[b], sem_ld)
        pltpu.async_copy(colh.at[pl.ds(base, C)], col_v.at[b], sem_ld)
        pltpu.async_copy(wbufh.at[pl.ds(base, C)], w_v.at[b], sem_ld)

    def issue_gathers(b):
        pltpu.make_async_copy(rowh.at[pl.ds(0, C)], row_v.at[b], sem_ld).wait()
        pltpu.make_async_copy(colh.at[pl.ds(0, C)], col_v.at[b], sem_ld).wait()
        pltpu.make_async_copy(wbufh.at[pl.ds(0, C)], w_v.at[b], sem_ld).wait()
        pltpu.async_copy(dinvh.at[row_v.at[b]], di_v.at[b], sem_d)
        pltpu.async_copy(xh.at[row_v.at[b]], a_v.at[pl.ds(b * C, C)], sem_a)

    issue_loads(0, 0)
    issue_gathers(0)

    def chunk(g, carry):
        b = g % 2
        nb = 1 - b

        @pl.when(g + 1 < nch)
        def _pref():
            issue_loads(g + 1, nb)

        pltpu.make_async_copy(dinvh.at[row_v.at[b]], di_v.at[b], sem_d).wait()
        for j in range(C // L):
            s = pl.ds(j * L, L)
            p_v[s] = w_v[b, s] * di_v[b, s]

        pltpu.make_async_copy(
            xh.at[row_v.at[b]], a_v.at[pl.ds(b * C, C)], sem_a).wait()

        def rowscale16(m, c2):
            p16 = p_v[pl.ds(m * L, L)]
            for jj in range(L):
                pv = p16[jj]
                j = b * C + m * L + jj
                for k in range(D // L):
                    s = pl.ds(k * L, L)
                    a_v[j, s] = a_v[j, s] * pv
            return c2

        lax.fori_loop(0, C // L, rowscale16, 0)
        pltpu.sync_copy(a_v.at[pl.ds(b * C, C)], out_sh.at[col_v.at[b]],
                        add=True)

        @pl.when(g + 1 < nch)
        def _next():
            issue_gathers(nb)

        return carry

    lax.fori_loop(0, nch, chunk, 0)
    plsc.subcore_barrier()

    @pl.when(sid == 0)
    def _out():
        pltpu.sync_copy(out_sh, out_p.at[cid])


def _sc_phase_b(x, row, col, wbuf, dinv):
    mesh = plsc.VectorSubcoreMesh(core_axis_name="c", subcore_axis_name="s")
    zerosnd = jnp.zeros((N, D), jnp.float32)
    return pl.kernel(
        _sc_b_body,
        out_type=jax.ShapeDtypeStruct((2, N, D), jnp.float32),
        mesh=mesh,
        scratch_types=[
            pltpu.VMEM((2, C), jnp.int32),
            pltpu.VMEM((2, C), jnp.int32),
            pltpu.VMEM((2, C), jnp.float32),
            pltpu.VMEM((2, C), jnp.float32),
            pltpu.VMEM((C,), jnp.float32),
            pltpu.VMEM((2 * C, D), jnp.float32),
            pltpu.VMEM_SHARED((N, D), jnp.float32),
            pltpu.SemaphoreType.DMA,
            pltpu.SemaphoreType.DMA,
            pltpu.SemaphoreType.DMA,
        ],
    )(x, row, col, wbuf, dinv, zerosnd)


def _sum_body(p_ref, o_ref):
    o_ref[...] = p_ref[0] + p_ref[1]


def _sum_partials(out_p):
    rows = 2000
    return pl.pallas_call(
        _sum_body,
        grid=(N // rows,),
        in_specs=[pl.BlockSpec((2, rows, D), lambda i: (0, i, 0))],
        out_specs=pl.BlockSpec((rows, D), lambda i: (i, 0)),
        out_shape=jax.ShapeDtypeStruct((N, D), jnp.float32),
    )(out_p)


def kernel(x, edge_index, beta):
    row = edge_index[0]
    col = edge_index[1]
    xnp = jnp.pad(_normalize(x), ((0, NP - N), (0, 0)))
    w = _build_w(xnp, beta)
    wbuf, denom_p = _sc_phase_a(w.reshape(NP * NP), row, col)
    out_p = _sc_phase_b(x, row, col, wbuf, _dinv(denom_p))
    return _sum_partials(out_p)


# speedup vs baseline: 13.2853x; 1.1279x over previous
import functools

import jax
import jax.numpy as jnp
from jax import lax
from jax.experimental import pallas as pl
from jax.experimental.pallas import tpu as pltpu
from jax.experimental.pallas import tpu_sc as plsc

N = 10000
D = 128
E = 320000

C = 128
NCHUNKS = E // C
NW = 32
L = 16

NP = 10240
BM = 2048
BN = 128


def _norm_body(x_ref, o_ref):
    xv = x_ref[...]
    s2 = jnp.sum(xv * xv, axis=1, keepdims=True)
    o_ref[...] = xv * lax.rsqrt(s2 + 1e-30)


def _normalize(x):
    return pl.pallas_call(
        _norm_body,
        out_shape=jax.ShapeDtypeStruct((N, D), jnp.float32),
    )(x)


def _w_body(beta_ref, a_ref, b_ref, w_ref):
    a = a_ref[...]
    b = b_ref[...]
    dn = (((1,), (1,)), ((), ()))
    cos = lax.dot_general(a, b, dn, preferred_element_type=jnp.float32)
    w_ref[...] = jnp.exp(beta_ref[0] * cos)[None]


def _w_ij(t):
    i = ((t >= 80).astype(jnp.int32) + (t >= 144).astype(jnp.int32)
         + (t >= 192).astype(jnp.int32) + (t >= 224).astype(jnp.int32))
    base = (jnp.where(i >= 1, 80, 0) + jnp.where(i >= 2, 64, 0)
            + jnp.where(i >= 3, 48, 0) + jnp.where(i >= 4, 32, 0))
    j = t - base + 16 * i
    return i, j


N_WBLOCKS = 240


def _build_w(x, beta):
    return pl.pallas_call(
        _w_body,
        grid=(N_WBLOCKS,),
        in_specs=[
            pl.BlockSpec(memory_space=pltpu.SMEM),
            pl.BlockSpec((BM, D), lambda t: (_w_ij(t)[0], 0)),
            pl.BlockSpec((BN, D), lambda t: (_w_ij(t)[1], 0)),
        ],
        out_specs=pl.BlockSpec(
            (1, BM, BN), lambda t: (_w_ij(t)[1], _w_ij(t)[0], 0)),
        out_shape=jax.ShapeDtypeStruct((NP // BN, NP, BN), jnp.float32),
    )(jnp.reshape(beta, (1,)), x, x)


def _sc_a_body(wflat, rowh, colh, zerosn,
               wbuf, denom_p,
               row_v, col_v, lin_v, w_v,
               denom_sh, sem_ld, sem_g):
    cid = lax.axis_index("c")
    sid = lax.axis_index("s")
    wid = cid * 16 + sid

    @pl.when(sid == 0)
    def _init():
        pltpu.sync_copy(zerosn, denom_sh)

    plsc.subcore_barrier()

    nch = (NCHUNKS - wid + (NW - 1)) // NW

    def issue_loads(g, b):
        base = (wid + g * NW) * C
        pltpu.async_copy(rowh.at[pl.ds(base, C)], row_v.at[b], sem_ld)
        pltpu.async_copy(colh.at[pl.ds(base, C)], col_v.at[b], sem_ld)

    def issue_gather(b):
        pltpu.make_async_copy(rowh.at[pl.ds(0, C)], row_v.at[b], sem_ld).wait()
        pltpu.make_async_copy(colh.at[pl.ds(0, C)], col_v.at[b], sem_ld).wait()
        for j in range(C // L):
            s = pl.ds(j * L, L)
            r = row_v[b, s]
            c = col_v[b, s]
            rm = jnp.minimum(r, c)
            cm = jnp.maximum(r, c)
            lin_v[b, s] = (cm >> 7) * (NP * 128) + (rm << 7) + (cm & 127)
        pltpu.async_copy(wflat.at[lin_v.at[b]], w_v.at[b], sem_g)

    issue_loads(0, 0)
    issue_gather(0)

    def chunk(g, carry):
        b = g % 2
        nb = 1 - b
        base = (wid + g * NW) * C

        @pl.when(g + 1 < nch)
        def _pref():
            issue_loads(g + 1, nb)

        pltpu.make_async_copy(
            wflat.at[lin_v.at[b]], w_v.at[b], sem_g).wait()
        pltpu.sync_copy(w_v.at[b], wbuf.at[pl.ds(base, C)])
        pltpu.sync_copy(w_v.at[b], denom_sh.at[row_v.at[b]], add=True)

        @pl.when(g + 1 < nch)
        def _next():
            issue_gather(nb)

        return carry

    lax.fori_loop(0, nch, chunk, 0)
    plsc.subcore_barrier()

    @pl.when(sid == 0)
    def _out():
        pltpu.sync_copy(denom_sh, denom_p.at[cid])


def _sc_phase_a(wflat, row, col):
    mesh = plsc.VectorSubcoreMesh(core_axis_name="c", subcore_axis_name="s")
    zerosn = jnp.zeros((N,), jnp.float32)
    return pl.kernel(
        _sc_a_body,
        out_type=(
            jax.ShapeDtypeStruct((E,), jnp.float32),
            jax.ShapeDtypeStruct((2, N), jnp.float32),
        ),
        mesh=mesh,
        scratch_types=[
            pltpu.VMEM((2, C), jnp.int32),
            pltpu.VMEM((2, C), jnp.int32),
            pltpu.VMEM((2, C), jnp.int32),
            pltpu.VMEM((2, C), jnp.float32),
            pltpu.VMEM_SHARED((N,), jnp.float32),
            pltpu.SemaphoreType.DMA,
            pltpu.SemaphoreType.DMA,
        ],
    )(wflat, row, col, zerosn)


def _dinv_body(p_ref, o_ref):
    o_ref[...] = 1.0 / (p_ref[0] + p_ref[1] + 1e-16)


def _dinv(denom_p):
    return pl.pallas_call(
        _dinv_body,
        out_shape=jax.ShapeDtypeStruct((N,), jnp.float32),
    )(denom_p)


def _sc_b_body(xh, rowh, colh, wbufh, dinvh, zerosnd,
               out_p,
               row_v, col_v, w_v, di_v, p_v, a_v,
               out_sh, sem_ld, sem_d, sem_a):
    cid = lax.axis_index("c")
    sid = lax.axis_index("s")
    wid = cid * 16 + sid

    @pl.when(sid == 0)
    def _init():
        pltpu.sync_copy(zerosnd, out_sh)

    plsc.subcore_barrier()

    nch = (NCHUNKS - wid + (NW - 1)) // NW

    def issue_loads(g, b):
        base = (wid + g * NW) * C
        pltpu.async_copy(rowh.at[pl.ds(base, C)], row_v.at[b], sem_ld)
        pltpu.async_copy(colh.at[pl.ds(base, C)], col_v.at[b], sem_ld)
        pltpu.async_copy(wbufh.at[pl.ds(base, C)], w_v.at[b], sem_ld)

    def issue_gathers(b):
        pltpu.make_async_copy(rowh.at[pl.ds(0, C)], row_v.at[b], sem_ld).wait()
        pltpu.make_async_copy(colh.at[pl.ds(0, C)], col_v.at[b], sem_ld).wait()
        pltpu.make_async_copy(wbufh.at[pl.ds(0, C)], w_v.at[b], sem_ld).wait()
        pltpu.async_copy(dinvh.at[row_v.at[b]], di_v.at[b], sem_d)
        pltpu.async_copy(xh.at[row_v.at[b]], a_v.at[pl.ds(b * C, C)], sem_a)

    issue_loads(0, 0)
    issue_gathers(0)

    def chunk(g, carry):
        b = g % 2
        nb = 1 - b

        @pl.when(g + 1 < nch)
        def _pref():
            issue_loads(g + 1, nb)

        pltpu.make_async_copy(dinvh.at[row_v.at[b]], di_v.at[b], sem_d).wait()
        for j in range(C // L):
            s = pl.ds(j * L, L)
            p_v[s] = w_v[b, s] * di_v[b, s]

        pltpu.make_async_copy(
            xh.at[row_v.at[b]], a_v.at[pl.ds(b * C, C)], sem_a).wait()

        def rowscale16(m, c2):
            p16 = p_v[pl.ds(m * L, L)]
            for jj in range(L):
                pv = p16[jj]
                j = b * C + m * L + jj
                for k in range(D // L):
                    s = pl.ds(k * L, L)
                    a_v[j, s] = a_v[j, s] * pv
            return c2

        lax.fori_loop(0, C // L, rowscale16, 0)
        pltpu.sync_copy(a_v.at[pl.ds(b * C, C)], out_sh.at[col_v.at[b]],
                        add=True)

        @pl.when(g + 1 < nch)
        def _next():
            issue_gathers(nb)

        return carry

    lax.fori_loop(0, nch, chunk, 0)
    plsc.subcore_barrier()

    @pl.when(sid == 0)
    def _out():
        pltpu.sync_copy(out_sh, out_p.at[cid])


def _sc_phase_b(x, row, col, wbuf, dinv):
    mesh = plsc.VectorSubcoreMesh(core_axis_name="c", subcore_axis_name="s")
    zerosnd = jnp.zeros((N, D), jnp.float32)
    return pl.kernel(
        _sc_b_body,
        out_type=jax.ShapeDtypeStruct((2, N, D), jnp.float32),
        mesh=mesh,
        scratch_types=[
            pltpu.VMEM((2, C), jnp.int32),
            pltpu.VMEM((2, C), jnp.int32),
            pltpu.VMEM((2, C), jnp.float32),
            pltpu.VMEM((2, C), jnp.float32),
            pltpu.VMEM((C,), jnp.float32),
            pltpu.VMEM((2 * C, D), jnp.float32),
            pltpu.VMEM_SHARED((N, D), jnp.float32),
            pltpu.SemaphoreType.DMA,
            pltpu.SemaphoreType.DMA,
            pltpu.SemaphoreType.DMA,
        ],
    )(x, row, col, wbuf, dinv, zerosnd)


def _sum_body(p_ref, o_ref):
    o_ref[...] = p_ref[0] + p_ref[1]


def _sum_partials(out_p):
    rows = 2000
    return pl.pallas_call(
        _sum_body,
        grid=(N // rows,),
        in_specs=[pl.BlockSpec((2, rows, D), lambda i: (0, i, 0))],
        out_specs=pl.BlockSpec((rows, D), lambda i: (i, 0)),
        out_shape=jax.ShapeDtypeStruct((N, D), jnp.float32),
    )(out_p)


def kernel(x, edge_index, beta):
    row = edge_index[0]
    col = edge_index[1]
    xnp = jnp.pad(_normalize(x), ((0, NP - N), (0, 0)))
    w = _build_w(xnp, beta)
    wbuf, denom_p = _sc_phase_a(w.reshape(NP * NP), row, col)
    out_p = _sc_phase_b(x, row, col, wbuf, _dinv(denom_p))
    return _sum_partials(out_p)


# speedup vs baseline: 14.4871x; 1.0905x over previous
import functools

import jax
import jax.numpy as jnp
from jax import lax
from jax.experimental import pallas as pl
from jax.experimental.pallas import tpu as pltpu
from jax.experimental.pallas import tpu_sc as plsc

N = 10000
D = 128
E = 320000

C = 128
NCHUNKS = E // C
NW = 32
L = 16

NP = 10240
BM = 2048
BN = 128


def _norm_body(x_ref, o_ref):
    xv = x_ref[...]
    s2 = jnp.sum(xv * xv, axis=1, keepdims=True)
    o_ref[...] = xv * lax.rsqrt(s2 + 1e-30)


def _normalize(x):
    return pl.pallas_call(
        _norm_body,
        out_shape=jax.ShapeDtypeStruct((N, D), jnp.float32),
    )(x)


def _w_body(beta_ref, a_ref, b_ref, w_ref):
    a = a_ref[...]
    b = b_ref[...]
    dn = (((1,), (1,)), ((), ()))
    cos = lax.dot_general(a, b, dn, preferred_element_type=jnp.float32)
    w_ref[...] = jnp.exp(beta_ref[0] * cos)[None]


def _w_ij(t):
    i = ((t >= 80).astype(jnp.int32) + (t >= 144).astype(jnp.int32)
         + (t >= 192).astype(jnp.int32) + (t >= 224).astype(jnp.int32))
    base = (jnp.where(i >= 1, 80, 0) + jnp.where(i >= 2, 64, 0)
            + jnp.where(i >= 3, 48, 0) + jnp.where(i >= 4, 32, 0))
    j = t - base + 16 * i
    return i, j


N_WBLOCKS = 240


def _build_w(x, beta):
    return pl.pallas_call(
        _w_body,
        grid=(N_WBLOCKS,),
        in_specs=[
            pl.BlockSpec(memory_space=pltpu.SMEM),
            pl.BlockSpec((BM, D), lambda t: (_w_ij(t)[0], 0)),
            pl.BlockSpec((BN, D), lambda t: (_w_ij(t)[1], 0)),
        ],
        out_specs=pl.BlockSpec(
            (1, BM, BN), lambda t: (_w_ij(t)[1], _w_ij(t)[0], 0)),
        out_shape=jax.ShapeDtypeStruct((NP // BN, NP, BN), jnp.float32),
    )(jnp.reshape(beta, (1,)), x, x)


def _sc_a_body(wflat, rowh, colh, zerosn,
               wbuf, denom_p,
               row_v, col_v, lin_v, w_v,
               denom_sh, sem_ld, sem_g):
    cid = lax.axis_index("c")
    sid = lax.axis_index("s")
    wid = cid * 16 + sid

    @pl.when(sid == 0)
    def _init():
        pltpu.sync_copy(zerosn, denom_sh)

    plsc.subcore_barrier()

    nch = (NCHUNKS - wid + (NW - 1)) // NW

    def issue_loads(g, b):
        base = (wid + g * NW) * C
        pltpu.async_copy(rowh.at[pl.ds(base, C)], row_v.at[b], sem_ld)
        pltpu.async_copy(colh.at[pl.ds(base, C)], col_v.at[b], sem_ld)

    def issue_gather(b):
        pltpu.make_async_copy(rowh.at[pl.ds(0, C)], row_v.at[b], sem_ld).wait()
        pltpu.make_async_copy(colh.at[pl.ds(0, C)], col_v.at[b], sem_ld).wait()
        for j in range(C // L):
            s = pl.ds(j * L, L)
            r = row_v[b, s]
            c = col_v[b, s]
            rm = jnp.minimum(r, c)
            cm = jnp.maximum(r, c)
            lin_v[b, s] = (cm >> 7) * (NP * 128) + (rm << 7) + (cm & 127)
        pltpu.async_copy(wflat.at[lin_v.at[b]], w_v.at[b], sem_g)

    issue_loads(0, 0)
    issue_gather(0)

    def chunk(g, carry):
        b = g % 2
        nb = 1 - b
        base = (wid + g * NW) * C

        @pl.when(g + 1 < nch)
        def _pref():
            issue_loads(g + 1, nb)

        pltpu.make_async_copy(
            wflat.at[lin_v.at[b]], w_v.at[b], sem_g).wait()
        pltpu.sync_copy(w_v.at[b], wbuf.at[pl.ds(base, C)])
        pltpu.sync_copy(w_v.at[b], denom_sh.at[row_v.at[b]], add=True)

        @pl.when(g + 1 < nch)
        def _next():
            issue_gather(nb)

        return carry

    lax.fori_loop(0, nch, chunk, 0)
    plsc.subcore_barrier()

    @pl.when(sid == 0)
    def _out():
        pltpu.sync_copy(denom_sh, denom_p.at[cid])


def _sc_phase_a(wflat, row, col):
    mesh = plsc.VectorSubcoreMesh(core_axis_name="c", subcore_axis_name="s")
    zerosn = jnp.zeros((N,), jnp.float32)
    return pl.kernel(
        _sc_a_body,
        out_type=(
            jax.ShapeDtypeStruct((E,), jnp.float32),
            jax.ShapeDtypeStruct((2, N), jnp.float32),
        ),
        mesh=mesh,
        scratch_types=[
            pltpu.VMEM((2, C), jnp.int32),
            pltpu.VMEM((2, C), jnp.int32),
            pltpu.VMEM((2, C), jnp.int32),
            pltpu.VMEM((2, C), jnp.float32),
            pltpu.VMEM_SHARED((N,), jnp.float32),
            pltpu.SemaphoreType.DMA,
            pltpu.SemaphoreType.DMA,
        ],
    )(wflat, row, col, zerosn)


def _dinv_body(p_ref, o_ref):
    o_ref[...] = 1.0 / (p_ref[0] + p_ref[1] + 1e-16)


def _dinv(denom_p):
    return pl.pallas_call(
        _dinv_body,
        out_shape=jax.ShapeDtypeStruct((N,), jnp.float32),
    )(denom_p)


def _sc_b_body(xh, rowh, colh, wbufh, dinvh, zerosnd,
               out_p,
               row_v, col_v, w_v, di_v, p_v, a_v,
               out_sh, sem_ld, sem_d, sem_a, sem_sc):
    cid = lax.axis_index("c")
    sid = lax.axis_index("s")
    wid = cid * 16 + sid

    @pl.when(sid == 0)
    def _init():
        pltpu.sync_copy(zerosnd, out_sh)

    plsc.subcore_barrier()

    nch = (NCHUNKS - wid + (NW - 1)) // NW

    def issue_loads(g, b):
        base = (wid + g * NW) * C
        pltpu.async_copy(rowh.at[pl.ds(base, C)], row_v.at[b], sem_ld)
        pltpu.async_copy(colh.at[pl.ds(base, C)], col_v.at[b], sem_ld)
        pltpu.async_copy(wbufh.at[pl.ds(base, C)], w_v.at[b], sem_ld)

    def wait_scatter(b):
        pltpu.make_async_copy(
            a_v.at[pl.ds(b * C, C)], out_sh.at[col_v.at[b]], sem_sc).wait()

    def issue_gathers(b):
        pltpu.make_async_copy(rowh.at[pl.ds(0, C)], row_v.at[b], sem_ld).wait()
        pltpu.make_async_copy(colh.at[pl.ds(0, C)], col_v.at[b], sem_ld).wait()
        pltpu.make_async_copy(wbufh.at[pl.ds(0, C)], w_v.at[b], sem_ld).wait()
        pltpu.async_copy(dinvh.at[row_v.at[b]], di_v.at[b], sem_d)
        pltpu.async_copy(xh.at[row_v.at[b]], a_v.at[pl.ds(b * C, C)], sem_a)

    issue_loads(0, 0)
    issue_gathers(0)

    def chunk(g, carry):
        b = g % 2
        nb = 1 - b

        @pl.when(g + 1 < nch)
        def _pref():
            issue_loads(g + 1, nb)

        pltpu.make_async_copy(dinvh.at[row_v.at[b]], di_v.at[b], sem_d).wait()
        for j in range(C // L):
            s = pl.ds(j * L, L)
            p_v[s] = w_v[b, s] * di_v[b, s]

        pltpu.make_async_copy(
            xh.at[row_v.at[b]], a_v.at[pl.ds(b * C, C)], sem_a).wait()

        def rowscale16(m, c2):
            p16 = p_v[pl.ds(m * L, L)]
            for jj in range(L):
                pv = p16[jj]
                j = b * C + m * L + jj
                for k in range(D // L):
                    s = pl.ds(k * L, L)
                    a_v[j, s] = a_v[j, s] * pv
            return c2

        lax.fori_loop(0, C // L, rowscale16, 0)
        pltpu.async_copy(a_v.at[pl.ds(b * C, C)], out_sh.at[col_v.at[b]],
                         sem_sc, add=True)

        @pl.when(g + 1 < nch)
        def _next():
            @pl.when(g >= 1)
            def _drain():
                wait_scatter(nb)

            issue_gathers(nb)

        return carry

    lax.fori_loop(0, nch, chunk, 0)
    wait_scatter((nch - 2) % 2)
    wait_scatter((nch - 1) % 2)
    plsc.subcore_barrier()

    @pl.when(sid == 0)
    def _out():
        pltpu.sync_copy(out_sh, out_p.at[cid])


def _sc_phase_b(x, row, col, wbuf, dinv):
    mesh = plsc.VectorSubcoreMesh(core_axis_name="c", subcore_axis_name="s")
    zerosnd = jnp.zeros((N, D), jnp.float32)
    return pl.kernel(
        _sc_b_body,
        out_type=jax.ShapeDtypeStruct((2, N, D), jnp.float32),
        mesh=mesh,
        scratch_types=[
            pltpu.VMEM((2, C), jnp.int32),
            pltpu.VMEM((2, C), jnp.int32),
            pltpu.VMEM((2, C), jnp.float32),
            pltpu.VMEM((2, C), jnp.float32),
            pltpu.VMEM((C,), jnp.float32),
            pltpu.VMEM((2 * C, D), jnp.float32),
            pltpu.VMEM_SHARED((N, D), jnp.float32),
            pltpu.SemaphoreType.DMA,
            pltpu.SemaphoreType.DMA,
            pltpu.SemaphoreType.DMA,
            pltpu.SemaphoreType.DMA,
        ],
    )(x, row, col, wbuf, dinv, zerosnd)


def _sum_body(p_ref, o_ref):
    o_ref[...] = p_ref[0] + p_ref[1]


def _sum_partials(out_p):
    rows = 2000
    return pl.pallas_call(
        _sum_body,
        grid=(N // rows,),
        in_specs=[pl.BlockSpec((2, rows, D), lambda i: (0, i, 0))],
        out_specs=pl.BlockSpec((rows, D), lambda i: (i, 0)),
        out_shape=jax.ShapeDtypeStruct((N, D), jnp.float32),
    )(out_p)


def kernel(x, edge_index, beta):
    row = edge_index[0]
    col = edge_index[1]
    xnp = jnp.pad(_normalize(x), ((0, NP - N), (0, 0)))
    w = _build_w(xnp, beta)
    wbuf, denom_p = _sc_phase_a(w.reshape(NP * NP), row, col)
    out_p = _sc_phase_b(x, row, col, wbuf, _dinv(denom_p))
    return _sum_partials(out_p)


# speedup vs baseline: 15.8052x; 1.0910x over previous
import functools

import jax
import jax.numpy as jnp
from jax import lax
from jax.experimental import pallas as pl
from jax.experimental.pallas import tpu as pltpu
from jax.experimental.pallas import tpu_sc as plsc

N = 10000
D = 128
E = 320000

C = 128
NCHUNKS = E // C
NW = 32
L = 16

NP = 10240
BM = 2048
BN = 128


def _norm_body(x_ref, o_ref):
    xv = x_ref[...]
    s2 = jnp.sum(xv * xv, axis=1, keepdims=True)
    o_ref[...] = xv * lax.rsqrt(s2 + 1e-30)


def _normalize(x):
    return pl.pallas_call(
        _norm_body,
        out_shape=jax.ShapeDtypeStruct((N, D), jnp.float32),
    )(x)


def _w_body(beta_ref, a_ref, b_ref, w_ref):
    a = a_ref[...]
    b = b_ref[...]
    dn = (((1,), (1,)), ((), ()))
    cos = lax.dot_general(a, b, dn, preferred_element_type=jnp.float32)
    w_ref[...] = jnp.exp(beta_ref[0] * cos)[None]


def _w_ij(t):
    i = ((t >= 80).astype(jnp.int32) + (t >= 144).astype(jnp.int32)
         + (t >= 192).astype(jnp.int32) + (t >= 224).astype(jnp.int32))
    base = (jnp.where(i >= 1, 80, 0) + jnp.where(i >= 2, 64, 0)
            + jnp.where(i >= 3, 48, 0) + jnp.where(i >= 4, 32, 0))
    j = t - base + 16 * i
    return i, j


N_WBLOCKS = 240


def _build_w(x, beta):
    return pl.pallas_call(
        _w_body,
        grid=(N_WBLOCKS,),
        in_specs=[
            pl.BlockSpec(memory_space=pltpu.SMEM),
            pl.BlockSpec((BM, D), lambda t: (_w_ij(t)[0], 0)),
            pl.BlockSpec((BN, D), lambda t: (_w_ij(t)[1], 0)),
        ],
        out_specs=pl.BlockSpec(
            (1, BM, BN), lambda t: (_w_ij(t)[1], _w_ij(t)[0], 0)),
        out_shape=jax.ShapeDtypeStruct((NP // BN, NP, BN), jnp.float32),
    )(jnp.reshape(beta, (1,)), x, x)


def _sc_a_body(wflat, rowh, colh, zerosn,
               wbuf, denom_p,
               row_v, col_v, lin_v, w_v,
               denom_sh, sem_ld, sem_g):
    cid = lax.axis_index("c")
    sid = lax.axis_index("s")
    wid = cid * 16 + sid

    @pl.when(sid == 0)
    def _init():
        pltpu.sync_copy(zerosn, denom_sh)

    plsc.subcore_barrier()

    nch = (NCHUNKS - wid + (NW - 1)) // NW

    def issue_loads(g, b):
        base = (wid + g * NW) * C
        pltpu.async_copy(rowh.at[pl.ds(base, C)], row_v.at[b], sem_ld)
        pltpu.async_copy(colh.at[pl.ds(base, C)], col_v.at[b], sem_ld)

    def issue_gather(b):
        pltpu.make_async_copy(rowh.at[pl.ds(0, C)], row_v.at[b], sem_ld).wait()
        pltpu.make_async_copy(colh.at[pl.ds(0, C)], col_v.at[b], sem_ld).wait()
        for j in range(C // L):
            s = pl.ds(j * L, L)
            r = row_v[b, s]
            c = col_v[b, s]
            rm = jnp.minimum(r, c)
            cm = jnp.maximum(r, c)
            lin_v[b, s] = (cm >> 7) * (NP * 128) + (rm << 7) + (cm & 127)
        pltpu.async_copy(wflat.at[lin_v.at[b]], w_v.at[b], sem_g)

    issue_loads(0, 0)
    issue_gather(0)

    def chunk(g, carry):
        b = g % 2
        nb = 1 - b
        base = (wid + g * NW) * C

        @pl.when(g + 1 < nch)
        def _pref():
            issue_loads(g + 1, nb)

        pltpu.make_async_copy(
            wflat.at[lin_v.at[b]], w_v.at[b], sem_g).wait()
        pltpu.sync_copy(w_v.at[b], wbuf.at[pl.ds(base, C)])
        pltpu.sync_copy(w_v.at[b], denom_sh.at[row_v.at[b]], add=True)

        @pl.when(g + 1 < nch)
        def _next():
            issue_gather(nb)

        return carry

    lax.fori_loop(0, nch, chunk, 0)
    plsc.subcore_barrier()

    @pl.when(sid == 0)
    def _out():
        pltpu.sync_copy(denom_sh, denom_p.at[cid])


def _sc_phase_a(wflat, row, col):
    mesh = plsc.VectorSubcoreMesh(core_axis_name="c", subcore_axis_name="s")
    zerosn = jnp.zeros((N,), jnp.float32)
    return pl.kernel(
        _sc_a_body,
        out_type=(
            jax.ShapeDtypeStruct((E,), jnp.float32),
            jax.ShapeDtypeStruct((2, N), jnp.float32),
        ),
        mesh=mesh,
        scratch_types=[
            pltpu.VMEM((2, C), jnp.int32),
            pltpu.VMEM((2, C), jnp.int32),
            pltpu.VMEM((2, C), jnp.int32),
            pltpu.VMEM((2, C), jnp.float32),
            pltpu.VMEM_SHARED((N,), jnp.float32),
            pltpu.SemaphoreType.DMA,
            pltpu.SemaphoreType.DMA,
        ],
    )(wflat, row, col, zerosn)


def _dinv_body(p_ref, o_ref):
    o_ref[...] = 1.0 / (p_ref[0] + p_ref[1] + 1e-16)


def _dinv(denom_p):
    return pl.pallas_call(
        _dinv_body,
        out_shape=jax.ShapeDtypeStruct((N,), jnp.float32),
    )(denom_p)


def _sc_b_body(xh, rowh, colh, wbufh, dinvh, zerosnd,
               out_p,
               row_v, col_v, w_v, di_v, p_v, a_v,
               out_sh, sem_ld, sem_d, sem_a, sem_sc):
    cid = lax.axis_index("c")
    sid = lax.axis_index("s")
    wid = cid * 16 + sid

    @pl.when(sid == 0)
    def _init():
        pltpu.sync_copy(zerosnd, out_sh)

    plsc.subcore_barrier()

    nch = (NCHUNKS - wid + (NW - 1)) // NW

    def issue_loads(g, b):
        base = (wid + g * NW) * C
        pltpu.async_copy(rowh.at[pl.ds(base, C)], row_v.at[b], sem_ld)
        pltpu.async_copy(colh.at[pl.ds(base, C)], col_v.at[b], sem_ld)
        pltpu.async_copy(wbufh.at[pl.ds(base, C)], w_v.at[b], sem_ld)

    def wait_scatter(b):
        pltpu.make_async_copy(
            a_v.at[pl.ds(b * C, C)], out_sh.at[col_v.at[b]], sem_sc).wait()

    def issue_gathers(b):
        pltpu.make_async_copy(rowh.at[pl.ds(0, C)], row_v.at[b], sem_ld).wait()
        pltpu.make_async_copy(colh.at[pl.ds(0, C)], col_v.at[b], sem_ld).wait()
        pltpu.make_async_copy(wbufh.at[pl.ds(0, C)], w_v.at[b], sem_ld).wait()
        pltpu.async_copy(dinvh.at[row_v.at[b]], di_v.at[b], sem_d)
        pltpu.async_copy(xh.at[row_v.at[b]], a_v.at[pl.ds(b * C, C)], sem_a)

    issue_loads(0, 0)
    issue_gathers(0)

    def chunk(g, carry):
        b = g % 2
        nb = 1 - b

        @pl.when(g + 1 < nch)
        def _pref():
            issue_loads(g + 1, nb)

        pltpu.make_async_copy(dinvh.at[row_v.at[b]], di_v.at[b], sem_d).wait()
        for j in range(C // L):
            s = pl.ds(j * L, L)
            p_v[s] = w_v[b, s] * di_v[b, s]

        pltpu.make_async_copy(
            xh.at[row_v.at[b]], a_v.at[pl.ds(b * C, C)], sem_a).wait()

        @pl.when(g + 1 < nch)
        def _next():
            @pl.when(g >= 1)
            def _drain():
                wait_scatter(nb)

            issue_gathers(nb)

        def rowscale16(m, c2):
            p16 = p_v[pl.ds(m * L, L)]
            for jj in range(L):
                pv = p16[jj]
                j = b * C + m * L + jj
                for k in range(D // L):
                    s = pl.ds(k * L, L)
                    a_v[j, s] = a_v[j, s] * pv
            return c2

        lax.fori_loop(0, C // L, rowscale16, 0)
        pltpu.async_copy(a_v.at[pl.ds(b * C, C)], out_sh.at[col_v.at[b]],
                         sem_sc, add=True)
        return carry

    lax.fori_loop(0, nch, chunk, 0)
    wait_scatter((nch - 2) % 2)
    wait_scatter((nch - 1) % 2)
    plsc.subcore_barrier()

    @pl.when(sid == 0)
    def _out():
        pltpu.sync_copy(out_sh, out_p.at[cid])


def _sc_phase_b(x, row, col, wbuf, dinv):
    mesh = plsc.VectorSubcoreMesh(core_axis_name="c", subcore_axis_name="s")
    zerosnd = jnp.zeros((N, D), jnp.float32)
    return pl.kernel(
        _sc_b_body,
        out_type=jax.ShapeDtypeStruct((2, N, D), jnp.float32),
        mesh=mesh,
        scratch_types=[
            pltpu.VMEM((2, C), jnp.int32),
            pltpu.VMEM((2, C), jnp.int32),
            pltpu.VMEM((2, C), jnp.float32),
            pltpu.VMEM((2, C), jnp.float32),
            pltpu.VMEM((C,), jnp.float32),
            pltpu.VMEM((2 * C, D), jnp.float32),
            pltpu.VMEM_SHARED((N, D), jnp.float32),
            pltpu.SemaphoreType.DMA,
            pltpu.SemaphoreType.DMA,
            pltpu.SemaphoreType.DMA,
            pltpu.SemaphoreType.DMA,
        ],
    )(x, row, col, wbuf, dinv, zerosnd)


def _sum_body(p_ref, o_ref):
    o_ref[...] = p_ref[0] + p_ref[1]


def _sum_partials(out_p):
    rows = 2000
    return pl.pallas_call(
        _sum_body,
        grid=(N // rows,),
        in_specs=[pl.BlockSpec((2, rows, D), lambda i: (0, i, 0))],
        out_specs=pl.BlockSpec((rows, D), lambda i: (i, 0)),
        out_shape=jax.ShapeDtypeStruct((N, D), jnp.float32),
    )(out_p)


def kernel(x, edge_index, beta):
    row = edge_index[0]
    col = edge_index[1]
    xnp = jnp.pad(_normalize(x), ((0, NP - N), (0, 0)))
    w = _build_w(xnp, beta)
    wbuf, denom_p = _sc_phase_a(w.reshape(NP * NP), row, col)
    out_p = _sc_phase_b(x, row, col, wbuf, _dinv(denom_p))
    return _sum_partials(out_p)


# speedup vs baseline: 15.8451x; 1.0025x over previous
import functools

import jax
import jax.numpy as jnp
from jax import lax
from jax.experimental import pallas as pl
from jax.experimental.pallas import tpu as pltpu
from jax.experimental.pallas import tpu_sc as plsc

N = 10000
D = 128
E = 320000

C = 128
NCHUNKS = E // C
NW = 32
L = 16

NP = 10240
BM = 2048
BN = 128


def _norm_body(x_ref, o_ref):
    xv = x_ref[...]
    s2 = jnp.sum(xv * xv, axis=1, keepdims=True)
    o_ref[...] = xv * lax.rsqrt(s2 + 1e-30)


def _normalize(x):
    return pl.pallas_call(
        _norm_body,
        out_shape=jax.ShapeDtypeStruct((N, D), jnp.float32),
    )(x)


def _w_body(beta_ref, a_ref, b_ref, w_ref):
    a = a_ref[...]
    b = b_ref[...]
    dn = (((1,), (1,)), ((), ()))
    cos = lax.dot_general(a, b, dn, preferred_element_type=jnp.float32)
    w_ref[...] = (beta_ref[0] * cos)[None]


def _w_ij(t):
    i = ((t >= 80).astype(jnp.int32) + (t >= 144).astype(jnp.int32)
         + (t >= 192).astype(jnp.int32) + (t >= 224).astype(jnp.int32))
    base = (jnp.where(i >= 1, 80, 0) + jnp.where(i >= 2, 64, 0)
            + jnp.where(i >= 3, 48, 0) + jnp.where(i >= 4, 32, 0))
    j = t - base + 16 * i
    return i, j


N_WBLOCKS = 240


def _build_w(x, beta):
    return pl.pallas_call(
        _w_body,
        grid=(N_WBLOCKS,),
        in_specs=[
            pl.BlockSpec(memory_space=pltpu.SMEM),
            pl.BlockSpec((BM, D), lambda t: (_w_ij(t)[0], 0)),
            pl.BlockSpec((BN, D), lambda t: (_w_ij(t)[1], 0)),
        ],
        out_specs=pl.BlockSpec(
            (1, BM, BN), lambda t: (_w_ij(t)[1], _w_ij(t)[0], 0)),
        out_shape=jax.ShapeDtypeStruct((NP // BN, NP, BN), jnp.float32),
    )(jnp.reshape(beta, (1,)), x, x)


def _sc_a_body(wflat, rowh, colh, zerosn,
               wbuf, denom_p,
               row_v, col_v, lin_v, w_v,
               denom_sh, sem_ld, sem_g):
    cid = lax.axis_index("c")
    sid = lax.axis_index("s")
    wid = cid * 16 + sid

    @pl.when(sid == 0)
    def _init():
        pltpu.sync_copy(zerosn, denom_sh)

    plsc.subcore_barrier()

    nch = (NCHUNKS - wid + (NW - 1)) // NW

    def issue_loads(g, b):
        base = (wid + g * NW) * C
        pltpu.async_copy(rowh.at[pl.ds(base, C)], row_v.at[b], sem_ld)
        pltpu.async_copy(colh.at[pl.ds(base, C)], col_v.at[b], sem_ld)

    def issue_gather(b):
        pltpu.make_async_copy(rowh.at[pl.ds(0, C)], row_v.at[b], sem_ld).wait()
        pltpu.make_async_copy(colh.at[pl.ds(0, C)], col_v.at[b], sem_ld).wait()
        for j in range(C // L):
            s = pl.ds(j * L, L)
            r = row_v[b, s]
            c = col_v[b, s]
            rm = jnp.minimum(r, c)
            cm = jnp.maximum(r, c)
            lin_v[b, s] = (cm >> 7) * (NP * 128) + (rm << 7) + (cm & 127)
        pltpu.async_copy(wflat.at[lin_v.at[b]], w_v.at[b], sem_g)

    issue_loads(0, 0)
    issue_gather(0)

    def chunk(g, carry):
        b = g % 2
        nb = 1 - b
        base = (wid + g * NW) * C

        @pl.when(g + 1 < nch)
        def _pref():
            issue_loads(g + 1, nb)

        pltpu.make_async_copy(
            wflat.at[lin_v.at[b]], w_v.at[b], sem_g).wait()
        for j in range(C // L):
            s = pl.ds(j * L, L)
            w_v[b, s] = jnp.exp(w_v[b, s])
        pltpu.sync_copy(w_v.at[b], wbuf.at[pl.ds(base, C)])
        pltpu.sync_copy(w_v.at[b], denom_sh.at[row_v.at[b]], add=True)

        @pl.when(g + 1 < nch)
        def _next():
            issue_gather(nb)

        return carry

    lax.fori_loop(0, nch, chunk, 0)
    plsc.subcore_barrier()

    @pl.when(sid == 0)
    def _out():
        pltpu.sync_copy(denom_sh, denom_p.at[cid])


def _sc_phase_a(wflat, row, col):
    mesh = plsc.VectorSubcoreMesh(core_axis_name="c", subcore_axis_name="s")
    zerosn = jnp.zeros((N,), jnp.float32)
    return pl.kernel(
        _sc_a_body,
        out_type=(
            jax.ShapeDtypeStruct((E,), jnp.float32),
            jax.ShapeDtypeStruct((2, N), jnp.float32),
        ),
        mesh=mesh,
        scratch_types=[
            pltpu.VMEM((2, C), jnp.int32),
            pltpu.VMEM((2, C), jnp.int32),
            pltpu.VMEM((2, C), jnp.int32),
            pltpu.VMEM((2, C), jnp.float32),
            pltpu.VMEM_SHARED((N,), jnp.float32),
            pltpu.SemaphoreType.DMA,
            pltpu.SemaphoreType.DMA,
        ],
    )(wflat, row, col, zerosn)


def _dinv_body(p_ref, o_ref):
    o_ref[...] = 1.0 / (p_ref[0] + p_ref[1] + 1e-16)


def _dinv(denom_p):
    return pl.pallas_call(
        _dinv_body,
        out_shape=jax.ShapeDtypeStruct((N,), jnp.float32),
    )(denom_p)


def _sc_b_body(xh, rowh, colh, wbufh, dinvh, zerosnd,
               out_p,
               row_v, col_v, w_v, di_v, p_v, a_v,
               out_sh, sem_ld, sem_d, sem_a, sem_sc):
    cid = lax.axis_index("c")
    sid = lax.axis_index("s")
    wid = cid * 16 + sid

    @pl.when(sid == 0)
    def _init():
        pltpu.sync_copy(zerosnd, out_sh)

    plsc.subcore_barrier()

    nch = (NCHUNKS - wid + (NW - 1)) // NW

    def issue_loads(g, b):
        base = (wid + g * NW) * C
        pltpu.async_copy(rowh.at[pl.ds(base, C)], row_v.at[b], sem_ld)
        pltpu.async_copy(colh.at[pl.ds(base, C)], col_v.at[b], sem_ld)
        pltpu.async_copy(wbufh.at[pl.ds(base, C)], w_v.at[b], sem_ld)

    def wait_scatter(b):
        pltpu.make_async_copy(
            a_v.at[pl.ds(b * C, C)], out_sh.at[col_v.at[b]], sem_sc).wait()

    def issue_gathers(b):
        pltpu.make_async_copy(rowh.at[pl.ds(0, C)], row_v.at[b], sem_ld).wait()
        pltpu.make_async_copy(colh.at[pl.ds(0, C)], col_v.at[b], sem_ld).wait()
        pltpu.make_async_copy(wbufh.at[pl.ds(0, C)], w_v.at[b], sem_ld).wait()
        pltpu.async_copy(dinvh.at[row_v.at[b]], di_v.at[b], sem_d)
        pltpu.async_copy(xh.at[row_v.at[b]], a_v.at[pl.ds(b * C, C)], sem_a)

    issue_loads(0, 0)
    issue_gathers(0)

    def chunk(g, carry):
        b = g % 2
        nb = 1 - b

        @pl.when(g + 1 < nch)
        def _pref():
            issue_loads(g + 1, nb)

        pltpu.make_async_copy(dinvh.at[row_v.at[b]], di_v.at[b], sem_d).wait()
        for j in range(C // L):
            s = pl.ds(j * L, L)
            p_v[s] = w_v[b, s] * di_v[b, s]

        pltpu.make_async_copy(
            xh.at[row_v.at[b]], a_v.at[pl.ds(b * C, C)], sem_a).wait()

        @pl.when(g + 1 < nch)
        def _next():
            @pl.when(g >= 1)
            def _drain():
                wait_scatter(nb)

            issue_gathers(nb)

        def rowscale16(m, c2):
            p16 = p_v[pl.ds(m * L, L)]
            for jj in range(L):
                pv = p16[jj]
                j = b * C + m * L + jj
                for k in range(D // L):
                    s = pl.ds(k * L, L)
                    a_v[j, s] = a_v[j, s] * pv
            return c2

        lax.fori_loop(0, C // L, rowscale16, 0)
        pltpu.async_copy(a_v.at[pl.ds(b * C, C)], out_sh.at[col_v.at[b]],
                         sem_sc, add=True)
        return carry

    lax.fori_loop(0, nch, chunk, 0)
    wait_scatter((nch - 2) % 2)
    wait_scatter((nch - 1) % 2)
    plsc.subcore_barrier()

    @pl.when(sid == 0)
    def _out():
        pltpu.sync_copy(out_sh, out_p.at[cid])


def _sc_phase_b(x, row, col, wbuf, dinv):
    mesh = plsc.VectorSubcoreMesh(core_axis_name="c", subcore_axis_name="s")
    zerosnd = jnp.zeros((N, D), jnp.float32)
    return pl.kernel(
        _sc_b_body,
        out_type=jax.ShapeDtypeStruct((2, N, D), jnp.float32),
        mesh=mesh,
        scratch_types=[
            pltpu.VMEM((2, C), jnp.int32),
            pltpu.VMEM((2, C), jnp.int32),
            pltpu.VMEM((2, C), jnp.float32),
            pltpu.VMEM((2, C), jnp.float32),
            pltpu.VMEM((C,), jnp.float32),
            pltpu.VMEM((2 * C, D), jnp.float32),
            pltpu.VMEM_SHARED((N, D), jnp.float32),
            pltpu.SemaphoreType.DMA,
            pltpu.SemaphoreType.DMA,
            pltpu.SemaphoreType.DMA,
            pltpu.SemaphoreType.DMA,
        ],
    )(x, row, col, wbuf, dinv, zerosnd)


def _sum_body(p_ref, o_ref):
    o_ref[...] = p_ref[0] + p_ref[1]


def _sum_partials(out_p):
    rows = 2000
    return pl.pallas_call(
        _sum_body,
        grid=(N // rows,),
        in_specs=[pl.BlockSpec((2, rows, D), lambda i: (0, i, 0))],
        out_specs=pl.BlockSpec((rows, D), lambda i: (i, 0)),
        out_shape=jax.ShapeDtypeStruct((N, D), jnp.float32),
    )(out_p)


def kernel(x, edge_index, beta):
    row = edge_index[0]
    col = edge_index[1]
    xnp = jnp.pad(_normalize(x), ((0, NP - N), (0, 0)))
    w = _build_w(xnp, beta)
    wbuf, denom_p = _sc_phase_a(w.reshape(NP * NP), row, col)
    out_p = _sc_phase_b(x, row, col, wbuf, _dinv(denom_p))
    return _sum_partials(out_p)
